# Initial kernel scaffold; baseline (speedup 1.0000x reference)
import jax
import jax.numpy as jnp
from jax.experimental import pallas as pl


def kernel(tokens, values, emb_a, emb_b, proj_w):
    raise NotImplementedError("write your pallas kernel here")


# speedup vs baseline: 2.5728x; 2.5728x over previous
import functools

import jax
import jax.numpy as jnp
import numpy as np
from jax import lax
from jax.experimental import pallas as pl
from jax.experimental.pallas import tpu as pltpu
from jax.experimental.pallas import tpu_sc as plsc

NFD = 4
MASK_ID = 1
SOS_ID = 2
EOS_ID = 3
EMB_DIM = 64
HALF = EMB_DIM // 2

_LANES = 16


def _sc_scan_body(tok_hbm, val_hbm, out_hbm, tok_v, val_v, out_v):
    info = plsc.get_sparse_core_info()
    nc = info.num_cores
    wid = lax.axis_index("s") * nc + lax.axis_index("c")
    num_workers = nc * info.num_subcores
    groups = tok_hbm.shape[0]
    g_per_w = groups // num_workers
    seq = tok_hbm.shape[2]

    zf = jnp.zeros((_LANES,), jnp.float32)
    zi = jnp.zeros((_LANES,), jnp.int32)
    one_i = jnp.full((_LANES,), 1, jnp.int32)
    mask_i = jnp.full((_LANES,), MASK_ID, jnp.int32)
    nfd_i = jnp.full((_LANES,), NFD, jnp.int32)
    sos_i = jnp.full((_LANES,), SOS_ID, jnp.int32)
    eos_i = jnp.full((_LANES,), EOS_ID, jnp.int32)

    for gi in range(g_per_w):
        g = wid * g_per_w + gi
        pltpu.sync_copy(tok_hbm.at[g], tok_v)
        pltpu.sync_copy(val_hbm.at[g], val_v)
        for i in range(2):

            def step(s, carry, i=i):
                unk, ck, ci = carry
                t = tok_v[i, s, :]
                vraw = val_v[i, s, :]
                special = t <= nfd_i
                tcv = jnp.where(special, t, nfd_i)
                v = jnp.where(special, zf, vraw)
                sm = special & (t != sos_i) & (t != eos_i)
                unk2 = unk | jnp.where(sm, one_i, zi)
                unkb = unk2 > zi
                ck2 = ck + v
                pk = jnp.where(unkb, zf, ck2)
                ci2 = jnp.where(sm, zf, ci + v)
                tk = jnp.where(unkb & (tcv == nfd_i), mask_i, tcv)
                out_v[i, s, 0, :] = pk
                out_v[i, s, 1, :] = ci2
                out_v[i, s, 2, :] = tk.astype(jnp.float32)
                out_v[i, s, 3, :] = tcv.astype(jnp.float32)
                return (unk2, ck2, ci2)

            lax.fori_loop(0, seq, step, (zi, zf, zf))
        pltpu.sync_copy(out_v, out_hbm.at[g])


def _sc_scan(tok_r, val_r):
    groups, _, seq, lanes = tok_r.shape
    mesh = plsc.VectorSubcoreMesh(core_axis_name="c", subcore_axis_name="s")
    fn = functools.partial(
        pl.kernel,
        mesh=mesh,
        out_type=jax.ShapeDtypeStruct((groups, 2, seq, 4, lanes), jnp.float32),
        scratch_types=[
            pltpu.VMEM((2, seq, lanes), jnp.int32),
            pltpu.VMEM((2, seq, lanes), jnp.float32),
            pltpu.VMEM((2, seq, 4, lanes), jnp.float32),
        ],
        compiler_params=pltpu.CompilerParams(use_tc_tiling_on_sc=False),
    )(_sc_scan_body)
    return fn(tok_r, val_r)


def _row_lookup(tab_ref, idx):
    acc = jnp.zeros((idx.shape[0], EMB_DIM), jnp.float32)
    for r in range(NFD + 1):
        acc = jnp.where(idx == jnp.float32(r), tab_ref[r : r + 1, :], acc)
    return acc


def _tc_dense_body(x_ref, ea_ref, eb_ref, w_ref, o_ref):
    x = x_ref[:, :]
    rows = x.shape[0]
    kk = lax.broadcasted_iota(jnp.int32, (1, HALF), 1).astype(jnp.float32)
    freqs = jnp.exp(kk * jnp.float32(-np.log(10000.0) / HALF))

    def sincos(p):
        ang = p * freqs
        return jnp.concatenate([jnp.sin(ang), jnp.cos(ang)], axis=1)

    p0 = x[:, 0:1]
    q0 = x[:, 1:2]
    tk0 = x[:, 2:3]
    t0 = x[:, 3:4]
    p1 = x[:, 4:5]
    q1 = x[:, 5:6]
    tk1 = x[:, 6:7]
    t1 = x[:, 7:8]

    e0 = sincos(p0) + _row_lookup(ea_ref, tk0)
    e1 = sincos(p1) + _row_lookup(eb_ref, tk1)
    e2 = sincos(q0) + _row_lookup(ea_ref, t0)
    e3 = sincos(q1) + _row_lookup(eb_ref, t1)
    feats = jnp.concatenate([e0, e1, e2, e3], axis=1)
    o_ref[:, :] = jnp.dot(feats, w_ref[:, :], preferred_element_type=jnp.float32)


def _tc_dense(packed, emb_a, emb_b, w_t, block_rows=1024):
    n = packed.shape[0]
    proj = w_t.shape[1]
    grid = n // block_rows
    return pl.pallas_call(
        _tc_dense_body,
        grid=(grid,),
        in_specs=[
            pl.BlockSpec((block_rows, 8), lambda i: (i, 0)),
            pl.BlockSpec((NFD + 1, EMB_DIM), lambda i: (0, 0)),
            pl.BlockSpec((NFD + 1, EMB_DIM), lambda i: (0, 0)),
            pl.BlockSpec((4 * EMB_DIM, proj), lambda i: (0, 0)),
        ],
        out_specs=pl.BlockSpec((block_rows, proj), lambda i: (i, 0)),
        out_shape=jax.ShapeDtypeStruct((n, proj), jnp.float32),
        compiler_params=pltpu.CompilerParams(
            dimension_semantics=("arbitrary",),
        ),
    )(packed, emb_a, emb_b, w_t)


def kernel(tokens, values, emb_a, emb_b, proj_w):
    b, s = tokens.shape[:2]
    groups = b // _LANES
    tok_r = tokens.reshape(groups, _LANES, s, 2).transpose(0, 3, 2, 1)
    val_r = values.reshape(groups, _LANES, s, 2).transpose(0, 3, 2, 1)
    sc_out = _sc_scan(tok_r, val_r)
    packed = sc_out.transpose(0, 4, 2, 1, 3).reshape(b * s, 8)
    out = _tc_dense(packed, emb_a, emb_b, proj_w.T)
    return out.reshape(b, s, proj_w.shape[0])


# speedup vs baseline: 11.3581x; 4.4147x over previous
import functools

import jax
import jax.numpy as jnp
import numpy as np
from jax import lax
from jax.experimental import pallas as pl
from jax.experimental.pallas import tpu as pltpu
from jax.experimental.pallas import tpu_sc as plsc

NFD = 4
MASK_ID = 1
SOS_ID = 2
EOS_ID = 3
EMB_DIM = 64
HALF = EMB_DIM // 2

_LANES = 16


def _sc_scan_body(tok_hbm, val_hbm, out_hbm, tok_v, val_v, out_v):
    info = plsc.get_sparse_core_info()
    nc = info.num_cores
    wid = lax.axis_index("s") * nc + lax.axis_index("c")
    num_workers = nc * info.num_subcores
    groups = tok_hbm.shape[0]
    g_per_w = groups // num_workers
    seq = tok_hbm.shape[2]

    zf = jnp.zeros((_LANES,), jnp.float32)
    zi = jnp.zeros((_LANES,), jnp.int32)
    one_i = jnp.full((_LANES,), 1, jnp.int32)
    mask_i = jnp.full((_LANES,), MASK_ID, jnp.int32)
    nfd_i = jnp.full((_LANES,), NFD, jnp.int32)
    sos_i = jnp.full((_LANES,), SOS_ID, jnp.int32)
    eos_i = jnp.full((_LANES,), EOS_ID, jnp.int32)

    for gi in range(g_per_w):
        g = wid * g_per_w + gi
        pltpu.sync_copy(tok_hbm.at[g], tok_v)
        pltpu.sync_copy(val_hbm.at[g], val_v)
        for i in range(2):

            def step(s, carry, i=i):
                unk, ck, ci = carry
                t = tok_v[i, s, :]
                vraw = val_v[i, s, :]
                special = t <= nfd_i
                tcv = jnp.where(special, t, nfd_i)
                v = jnp.where(special, zf, vraw)
                sm = special & (t != sos_i) & (t != eos_i)
                unk2 = unk | jnp.where(sm, one_i, zi)
                unkb = unk2 > zi
                ck2 = ck + v
                pk = jnp.where(unkb, zf, ck2)
                ci2 = jnp.where(sm, zf, ci + v)
                tk = jnp.where(unkb & (tcv == nfd_i), mask_i, tcv)
                out_v[i, s, 0, :] = pk
                out_v[i, s, 1, :] = ci2
                out_v[i, s, 2, :] = tk.astype(jnp.float32)
                out_v[i, s, 3, :] = tcv.astype(jnp.float32)
                return (unk2, ck2, ci2)

            lax.fori_loop(0, seq, step, (zi, zf, zf))
        pltpu.sync_copy(out_v, out_hbm.at[g])


def _sc_scan(tok_r, val_r):
    groups, _, seq, lanes = tok_r.shape
    mesh = plsc.VectorSubcoreMesh(core_axis_name="c", subcore_axis_name="s")
    fn = functools.partial(
        pl.kernel,
        mesh=mesh,
        out_type=jax.ShapeDtypeStruct((groups, 2, seq, 4, lanes), jnp.float32),
        scratch_types=[
            pltpu.VMEM((2, seq, lanes), jnp.int32),
            pltpu.VMEM((2, seq, lanes), jnp.float32),
            pltpu.VMEM((2, seq, 4, lanes), jnp.float32),
        ],
        compiler_params=pltpu.CompilerParams(use_tc_tiling_on_sc=False),
    )(_sc_scan_body)
    return fn(tok_r, val_r)


_INV2PI = 0.15915493667125702
_TWOPI = 6.2831854820251465
_S1, _S2, _S3, _S4, _S5 = (
    0.9999846160704663,
    -0.16663261875795207,
    0.008312396647128057,
    -0.0001931637862847349,
    2.1733051646932733e-06,
)
_C0, _C1, _C2, _C3, _C4 = (
    0.9999710932183866,
    -0.49983759608552286,
    0.04152230455014086,
    -0.0013441068677407103,
    1.906521608691092e-05,
)


def _fast_sincos(ang):
    n = jnp.floor(ang * _INV2PI + 0.5)
    r = ang - n * _TWOPI
    r2 = r * r
    s = r * (_S1 + r2 * (_S2 + r2 * (_S3 + r2 * (_S4 + r2 * _S5))))
    c = _C0 + r2 * (_C1 + r2 * (_C2 + r2 * (_C3 + r2 * _C4)))
    return s, c


def _tc_dense_body(x_ref, ea_ref, eb_ref, wp_ref, w_ref, o_ref, feats_ref, tab_ref):
    x = x_ref[:, :]

    colf = lax.broadcasted_iota(jnp.int32, (8, 4 * HALF), 1)
    rowi = lax.broadcasted_iota(jnp.int32, (8, 4 * HALF), 0)
    blk = colf // HALF
    kmod = (colf - blk * HALF).astype(jnp.float32)
    fr = jnp.exp(kmod * jnp.float32(-np.log(10000.0) / HALF))
    rowneed = jnp.where(
        blk == 0, 0, jnp.where(blk == 1, 4, jnp.where(blk == 2, 1, 5))
    )
    fmat = jnp.where(rowi == rowneed, fr, jnp.float32(0.0))
    ang = jnp.dot(x, fmat, preferred_element_type=jnp.float32)
    s_all, c_all = _fast_sincos(ang)
    feats_ref[:, 0 : 4 * HALF] = s_all
    feats_ref[:, 4 * HALF : 8 * HALF] = c_all

    tab_ref[:, :] = jnp.zeros((4 * (NFD + 1), 4 * EMB_DIM), jnp.float32)
    tab_ref[0:5, 0:64] = ea_ref[:, :]
    tab_ref[5:10, 64:128] = eb_ref[:, :]
    tab_ref[10:15, 128:192] = ea_ref[:, :]
    tab_ref[15:20, 192:256] = eb_ref[:, :]
    tp = jnp.dot(tab_ref[:, :], w_ref[:, :], preferred_element_type=jnp.float32)

    riota = lax.broadcasted_iota(jnp.int32, (1, NFD + 1), 1).astype(jnp.float32)
    ohs = [
        jnp.where(x[:, c : c + 1] == riota, jnp.float32(1.0), jnp.float32(0.0))
        for c in (2, 6, 3, 7)
    ]
    oh = jnp.concatenate(ohs, axis=1)

    o_ref[:, :] = jnp.dot(
        feats_ref[:, :], wp_ref[:, :], preferred_element_type=jnp.float32
    ) + jnp.dot(oh, tp, preferred_element_type=jnp.float32)


def _tc_dense(packed, emb_a, emb_b, w_p, w_t, block_rows=1024):
    n = packed.shape[0]
    proj = w_t.shape[1]
    grid = n // block_rows
    return pl.pallas_call(
        _tc_dense_body,
        grid=(grid,),
        in_specs=[
            pl.BlockSpec((block_rows, 8), lambda i: (i, 0)),
            pl.BlockSpec((NFD + 1, EMB_DIM), lambda i: (0, 0)),
            pl.BlockSpec((NFD + 1, EMB_DIM), lambda i: (0, 0)),
            pl.BlockSpec((4 * EMB_DIM, proj), lambda i: (0, 0)),
            pl.BlockSpec((4 * EMB_DIM, proj), lambda i: (0, 0)),
        ],
        out_specs=pl.BlockSpec((block_rows, proj), lambda i: (i, 0)),
        out_shape=jax.ShapeDtypeStruct((n, proj), jnp.float32),
        scratch_shapes=[
            pltpu.VMEM((block_rows, 4 * EMB_DIM), jnp.float32),
            pltpu.VMEM((4 * (NFD + 1), 4 * EMB_DIM), jnp.float32),
        ],
        compiler_params=pltpu.CompilerParams(
            dimension_semantics=("arbitrary",),
        ),
    )(packed, emb_a, emb_b, w_p, w_t)


def kernel(tokens, values, emb_a, emb_b, proj_w):
    b, s = tokens.shape[:2]
    groups = b // _LANES
    tok_r = tokens.reshape(groups, _LANES, s, 2).transpose(0, 3, 2, 1)
    val_r = values.reshape(groups, _LANES, s, 2).transpose(0, 3, 2, 1)
    sc_out = _sc_scan(tok_r, val_r)
    packed = sc_out.transpose(0, 4, 2, 1, 3).reshape(b * s, 8)
    w_t = proj_w.T
    w_p = w_t.reshape(4, 2, HALF, -1).transpose(1, 0, 2, 3).reshape(4 * EMB_DIM, -1)
    out = _tc_dense(packed, emb_a, emb_b, w_p, w_t)
    return out.reshape(b, s, proj_w.shape[0])
